# baseline (device time: 12982 ns/iter reference)
import jax
import jax.numpy as jnp
from jax import lax
from jax.experimental import pallas as pl
from jax.experimental.pallas import tpu as pltpu

N_DEV = 4


def kernel(x, w_mat):
    m_global, k_per = x.shape
    k_global, n = w_mat.shape
    m_per = m_global // N_DEV

    def body(x_ref, w_ref, out_ref, xbf_ref, comm_ref, send_sems, recv_sems):
        my = lax.axis_index("i")

        xbf_ref[:, :] = x_ref[:, :].astype(jnp.bfloat16)

        barrier_sem = pltpu.get_barrier_semaphore()
        for off in range(1, N_DEV):
            peer = (my + off) % N_DEV
            pl.semaphore_signal(
                barrier_sem, inc=1, device_id=(peer,),
                device_id_type=pl.DeviceIdType.MESH,
            )
        pl.semaphore_wait(barrier_sem, N_DEV - 1)

        sends = []
        for off in range(1, N_DEV):
            peer = (my + off) % N_DEV
            slot = N_DEV - off - 1
            rdma = pltpu.make_async_remote_copy(
                src_ref=xbf_ref.at[pl.ds(peer * m_per, m_per), :],
                dst_ref=comm_ref.at[slot],
                send_sem=send_sems.at[off - 1],
                recv_sem=recv_sems.at[slot],
                device_id=(peer,),
                device_id_type=pl.DeviceIdType.MESH,
            )
            rdma.start()
            sends.append(rdma)

        out_ref[:, :] = jnp.dot(
            xbf_ref[pl.ds(my * m_per, m_per), :],
            w_ref[pl.ds(my * k_per, k_per), :].astype(jnp.bfloat16),
            preferred_element_type=jnp.float32,
        )

        for r in range(1, N_DEV):
            recv = pltpu.make_async_remote_copy(
                src_ref=xbf_ref.at[pl.ds(0, m_per), :],
                dst_ref=comm_ref.at[r - 1],
                send_sem=send_sems.at[0],
                recv_sem=recv_sems.at[r - 1],
                device_id=(my,),
                device_id_type=pl.DeviceIdType.MESH,
            )
            recv.wait_recv()
            src = (my + r) % N_DEV
            out_ref[:, :] += jnp.dot(
                comm_ref[r - 1],
                w_ref[pl.ds(src * k_per, k_per), :].astype(jnp.bfloat16),
                preferred_element_type=jnp.float32,
            )

        for rdma in sends:
            rdma.wait_send()

    return pl.pallas_call(
        body,
        out_shape=jax.ShapeDtypeStruct((m_per, n), jnp.float32),
        in_specs=[
            pl.BlockSpec(memory_space=pltpu.VMEM),
            pl.BlockSpec(memory_space=pltpu.VMEM),
        ],
        out_specs=pl.BlockSpec(memory_space=pltpu.VMEM),
        scratch_shapes=[
            pltpu.VMEM((m_global, k_per), jnp.bfloat16),
            pltpu.VMEM((N_DEV - 1, m_per, k_per), jnp.bfloat16),
            pltpu.SemaphoreType.DMA((N_DEV - 1,)),
            pltpu.SemaphoreType.DMA((N_DEV - 1,)),
        ],
        compiler_params=pltpu.CompilerParams(collective_id=0),
    )(x, w_mat)
